# baseline (speedup 1.0000x reference)
import jax
import jax.numpy as jnp
from jax import lax
from jax.experimental import pallas as pl
from jax.experimental.pallas import tpu as pltpu
from jax.experimental.pallas import tpu_sc as plsc

B = 16384
F = 26
FP = 32
V = 100000
VPAD = 100096
D = 16

NC = 2
NS = 16
NW = NC * NS
EPW = B // NW
CE = 32
NCHUNK = EPW // CE
SR = F * VPAD // 8
AROWS = F * CE * 16 // 128


def _body(tab_hbm, sup_hbm, addr_hbm, out_hbm,
          sup_v, addr_v, srow_v, out_v, sem):
  c = lax.axis_index("c")
  s = lax.axis_index("s")
  wid = s * NC + c

  def chunk(ch, carry):
    pltpu.sync_copy(sup_hbm.at[wid, ch], sup_v)
    pltpu.sync_copy(addr_hbm.at[wid, ch], addr_v)

    descs = []
    for f in range(F):
      descs.append(pltpu.async_copy(
          tab_hbm.at[sup_v.at[f // 4, pl.ds((f % 4) * CE, CE)]],
          srow_v.at[pl.ds(f * CE, CE)], sem))
    for d in descs:
      d.wait()

    for j in range(CE):
      acc = plsc.load_gather(
          srow_v, [jnp.full((16,), j, jnp.int32),
                   addr_v[j // 8, pl.ds((j % 8) * 16, 16)]])
      for f in range(1, F):
        r = f * CE + j
        acc = acc + plsc.load_gather(
            srow_v, [jnp.full((16,), r, jnp.int32),
                     addr_v[r // 8, pl.ds((r % 8) * 16, 16)]])
      out_v[ch * 4 + j // 8, pl.ds((j % 8) * 16, 16)] = acc
    return carry

  lax.fori_loop(0, NCHUNK, chunk, 0)
  base = pl.multiple_of(wid * (EPW // 8), EPW // 8)
  pltpu.sync_copy(out_v, out_hbm.at[pl.ds(base, EPW // 8)])


@jax.jit
def _embed_sum(tab128, sup4, addr4):
  mesh = plsc.VectorSubcoreMesh(core_axis_name="c", subcore_axis_name="s")
  return pl.kernel(
      _body,
      out_type=jax.ShapeDtypeStruct((B * D // 128, 128), jnp.float32),
      mesh=mesh,
      scratch_types=[
          pltpu.VMEM((8, 128), jnp.int32),
          pltpu.VMEM((AROWS, 128), jnp.int32),
          pltpu.VMEM((F * CE, 128), jnp.float32),
          pltpu.VMEM((EPW // 8, 128), jnp.float32),
          pltpu.SemaphoreType.DMA,
      ],
      compiler_params=pltpu.CompilerParams(needs_layout_passes=False),
  )(tab128, sup4, addr4)


def kernel(x, tables):
  tab_p = jnp.pad(tables, ((0, 0), (0, VPAD - V), (0, 0)))
  tab128 = tab_p.reshape(SR, 128)
  offs = (jnp.arange(F, dtype=jnp.int32) * VPAD)[None, :]
  idx = x + offs
  sup = idx >> 3
  sup_p = jnp.pad(sup, ((0, 0), (0, FP - F)))
  col = ((idx & 7) << 4)[:, :, None] + jnp.arange(16, dtype=jnp.int32)
  sup4 = sup_p.reshape(NW, NCHUNK, CE, FP).transpose(0, 1, 3, 2) \
              .reshape(NW, NCHUNK, 8, 128)
  addr4 = col.reshape(NW, NCHUNK, CE, F, 16).transpose(0, 1, 3, 2, 4) \
             .reshape(NW, NCHUNK, AROWS, 128)
  return _embed_sum(tab128, sup4, addr4).reshape(B, D)


# speedup vs baseline: 1.1108x; 1.1108x over previous
import jax
import jax.numpy as jnp
from jax import lax
from jax.experimental import pallas as pl
from jax.experimental.pallas import tpu as pltpu
from jax.experimental.pallas import tpu_sc as plsc

B = 16384
F = 26
V = 100000
VPAD = 100096
D = 16

NC = 2
NS = 16
NW = NC * NS
EPW = B // NW
CE = 128
NCHUNK = EPW // CE


def _body(tab_hbm, idx_hbm, out_hbm, idx_v, rows_v, out_v, sem):
  c = lax.axis_index("c")
  s = lax.axis_index("s")
  wid = s * NC + c

  pltpu.sync_copy(idx_hbm.at[wid], idx_v)

  for ch in range(NCHUNK):
    descs = []
    for f in range(F):
      descs.append(pltpu.async_copy(
          tab_hbm.at[idx_v.at[f, pl.ds(ch * CE, CE)]],
          rows_v.at[f],
          sem,
      ))
    for d in descs:
      d.wait()

    def red(e, carry):
      acc = rows_v[0, e, :]
      for f in range(1, F):
        acc = acc + rows_v[f, e, :]
      out_v[e, :] = acc
      return carry

    lax.fori_loop(0, CE, red, 0)
    pltpu.sync_copy(out_v, out_hbm.at[pl.ds(wid * EPW + ch * CE, CE)])


@jax.jit
def _embed_sum(tab_flat, idx3):
  mesh = plsc.VectorSubcoreMesh(core_axis_name="c", subcore_axis_name="s")
  return pl.kernel(
      _body,
      out_type=jax.ShapeDtypeStruct((B, D), jnp.float32),
      mesh=mesh,
      scratch_types=[
          pltpu.VMEM((F, EPW), jnp.int32),
          pltpu.VMEM((F, CE, D), jnp.float32),
          pltpu.VMEM((CE, D), jnp.float32),
          pltpu.SemaphoreType.DMA,
      ],
      compiler_params=pltpu.CompilerParams(use_tc_tiling_on_sc=False),
  )(tab_flat, idx3)


def kernel(x, tables):
  tab_flat = jnp.pad(tables, ((0, 0), (0, VPAD - V), (0, 0))) \
                .reshape(F * VPAD, D)
  offs = (jnp.arange(F, dtype=jnp.int32) * VPAD)[None, :]
  idx = x + offs
  idx3 = idx.reshape(NW, EPW, F).transpose(0, 2, 1)
  return _embed_sum(tab_flat, idx3)


# speedup vs baseline: 1.7893x; 1.6108x over previous
import jax
import jax.numpy as jnp
from jax import lax
from jax.experimental import pallas as pl
from jax.experimental.pallas import tpu as pltpu
from jax.experimental.pallas import tpu_sc as plsc

B = 16384
F = 26
V = 100000
VPAD = 100096
D = 16

NC = 2
NS = 16
NW = NC * NS
EPW = B // NW
CE = 128
NCHUNK = EPW // CE


def _body(tab_hbm, idx_hbm, out_hbm, idx_v, rows_v, out_v, sem):
  c = lax.axis_index("c")
  s = lax.axis_index("s")
  wid = s * NC + c

  pltpu.sync_copy(idx_hbm.at[wid], idx_v)

  for ch in range(NCHUNK):
    descs = []
    for f in range(F):
      descs.append(pltpu.async_copy(
          tab_hbm.at[idx_v.at[f, pl.ds(ch * CE, CE)]],
          rows_v.at[f],
          sem,
      ))
    for d in descs:
      d.wait()

    def red(e, carry):
      acc = rows_v[0, e, :]
      for f in range(1, F):
        acc = acc + rows_v[f, e, :]
      out_v[e, :] = acc
      return carry

    lax.fori_loop(0, CE, red, 0)
    pltpu.sync_copy(out_v, out_hbm.at[pl.ds(wid * EPW + ch * CE, CE)])


@jax.jit
def _embed_sum(tab_flat, idx3):
  mesh = plsc.VectorSubcoreMesh(core_axis_name="c", subcore_axis_name="s")
  return pl.kernel(
      _body,
      out_type=jax.ShapeDtypeStruct((B, D), jnp.float32),
      mesh=mesh,
      scratch_types=[
          pltpu.VMEM((F, EPW), jnp.int32),
          pltpu.VMEM((F, CE, D), jnp.float32),
          pltpu.VMEM((CE, D), jnp.float32),
          pltpu.SemaphoreType.DMA,
      ],
      compiler_params=pltpu.CompilerParams(use_tc_tiling_on_sc=False),
  )(tab_flat, idx3)


def kernel(x, tables):
  zero = lax.optimization_barrier(jnp.float32(0.0))
  tab_flat = tables.reshape(F * V, D) + zero
  offs = (jnp.arange(F, dtype=jnp.int32) * V)[None, :]
  idx = x + offs
  idx3 = idx.reshape(NW, EPW, F).transpose(0, 2, 1)
  return _embed_sum(tab_flat, idx3)
